# baseline (speedup 1.0000x reference)
import jax
import jax.numpy as jnp
from jax import lax
from jax.experimental import pallas as pl
from jax.experimental.pallas import tpu as pltpu
from jax.experimental.pallas import tpu_sc as plsc

N = 10000
E = 320000
H = 128
NC = 2
NS = 16
NW = NC * NS
EPW = E // NW
GC = 80
SC_CHUNK = 200
NPT = 624
NTAIL = N - NPT * NS
EB = 2000
NB = 1000


def _gelu(x):
    return 0.5 * x * (1.0 + lax.erf(x * 0.7071067811865476))


def _prep_body(hv_ref, wp_ref, wq_ref, a2_ref, p_ref, q_ref, as_ref, ad_ref):
    hv = hv_ref[...]
    p_ref[...] = jnp.dot(hv, wp_ref[...], preferred_element_type=jnp.float32)
    q_ref[...] = jnp.dot(hv, wq_ref[...], preferred_element_type=jnp.float32)
    al = lax.dot_general(a2_ref[...], hv, (((1,), (1,)), ((), ())),
                         preferred_element_type=jnp.float32)
    as_ref[...] = al[0:1, :]
    ad_ref[...] = al[1:2, :]


def _edge_body(he_ref, g1_ref, g2_ref, lp_ref, w1e_ref, b1_ref, w2_ref,
               b2_ref, w3_ref, b3_ref, ae_ref, msg_ref, sum_ref):
    bf16 = jnp.bfloat16
    he = he_ref[...]
    x1 = (g1_ref[...] + g2_ref[...] + b1_ref[...]
          + jnp.dot(he.astype(bf16), w1e_ref[...].astype(bf16),
                    preferred_element_type=jnp.float32))
    x2 = jnp.dot(_gelu(x1).astype(bf16), w2_ref[...].astype(bf16),
                 preferred_element_type=jnp.float32) + b2_ref[...]
    msg = jnp.dot(_gelu(x2).astype(bf16), w3_ref[...].astype(bf16),
                  preferred_element_type=jnp.float32) + b3_ref[...]
    logit = lp_ref[...] + jnp.sum(he * ae_ref[...], axis=1, keepdims=True)
    leaky = jnp.where(logit >= 0, logit, 0.01 * logit)
    w = jnp.exp(1.0 / (1.0 + jnp.exp(-leaky)))

    @pl.when(pl.program_id(0) == 0)
    def _():
        sum_ref[0, 0] = 0.0

    sum_ref[0, 0] += jnp.sum(w)
    msg_ref[...] = msg * w


def _node_body(hv_ref, pa_ref, pb_ref, sum_ref, ln1g_ref, ln1b_ref, ln2g_ref,
               ln2b_ref, win_ref, winb_ref, wout_ref, woutb_ref, out_ref):
    scale = 1.0 / (30.0 * sum_ref[0, 0])
    x = hv_ref[...] + (pa_ref[...] + pb_ref[...]) * scale
    mu = jnp.mean(x, axis=1, keepdims=True)
    xc = x - mu
    var = jnp.mean(xc * xc, axis=1, keepdims=True)
    xn = xc * lax.rsqrt(var + 1e-5) * ln1g_ref[...] + ln1b_ref[...]
    y = jnp.dot(_gelu(jnp.dot(xn, win_ref[...], preferred_element_type=jnp.float32)
                      + winb_ref[...]),
                wout_ref[...], preferred_element_type=jnp.float32) + woutb_ref[...]
    z = xn + y
    mu2 = jnp.mean(z, axis=1, keepdims=True)
    zc = z - mu2
    var2 = jnp.mean(zc * zc, axis=1, keepdims=True)
    out_ref[...] = zc * lax.rsqrt(var2 + 1e-5) * ln2g_ref[...] + ln2b_ref[...]


def _sc_gather_body(p_hbm, q_hbm, as_hbm, ad_hbm, src_hbm, dst_hbm, g1_hbm,
                    g2_hbm, l_hbm, si_v, di_v, ps_v, qd_v, as_v, ad_v, gl_v,
                    sem1, sem2):
    wid = lax.axis_index("s") * NC + lax.axis_index("c")
    pltpu.sync_copy(as_hbm.at[0], as_v)
    pltpu.sync_copy(ad_hbm.at[0], ad_v)

    def chunk(i, carry):
        base = wid * EPW + i * GC
        pltpu.sync_copy(src_hbm.at[pl.ds(base, GC)], si_v)
        pltpu.sync_copy(dst_hbm.at[pl.ds(base, GC)], di_v)
        cp1 = pltpu.async_copy(p_hbm.at[si_v], ps_v, sem1)
        cp2 = pltpu.async_copy(q_hbm.at[di_v], qd_v, sem2)

        def lrow(r, c2):
            sl = pl.ds(r * 16, 16)
            va = plsc.load_gather(as_v, [si_v[sl]])
            vb = plsc.load_gather(ad_v, [di_v[sl]])
            gl_v[sl] = va + vb
            return c2

        lax.fori_loop(0, GC // 16, lrow, 0)
        cp1.wait()
        cp2.wait()
        pltpu.sync_copy(ps_v, g1_hbm.at[pl.ds(base, GC)])
        pltpu.sync_copy(qd_v, g2_hbm.at[pl.ds(base, GC)])
        pltpu.sync_copy(gl_v, l_hbm.at[pl.ds(base, GC)])
        return carry

    lax.fori_loop(0, EPW // GC, chunk, 0)


def _sc_scatter_body(msg_hbm, src_hbm, out_hbm, idx_v, msg_v, acc_sh, sem):
    cid = lax.axis_index("c")
    sid = lax.axis_index("s")
    wid = sid * NC + cid

    def zrow(r, c):
        for k in range(H // 16):
            msg_v[r, pl.ds(k * 16, 16)] = jnp.zeros((16,), jnp.float32)
        return c

    lax.fori_loop(0, SC_CHUNK, zrow, 0)
    for j in range(NPT // SC_CHUNK):
        pltpu.sync_copy(msg_v, acc_sh.at[pl.ds(sid * NPT + j * SC_CHUNK, SC_CHUNK)])
    pltpu.sync_copy(msg_v.at[pl.ds(0, NPT % SC_CHUNK)],
                    acc_sh.at[pl.ds(sid * NPT + (NPT // SC_CHUNK) * SC_CHUNK,
                                    NPT % SC_CHUNK)])

    @pl.when(sid == 0)
    def _():
        pltpu.sync_copy(msg_v.at[pl.ds(0, NTAIL)],
                        acc_sh.at[pl.ds(NPT * NS, NTAIL)])

    plsc.subcore_barrier()

    def chunk(i, carry):
        base = wid * EPW + i * SC_CHUNK
        pltpu.sync_copy(src_hbm.at[pl.ds(base, SC_CHUNK)], idx_v)
        pltpu.sync_copy(msg_hbm.at[pl.ds(base, SC_CHUNK)], msg_v)
        pltpu.sync_copy(msg_v, acc_sh.at[idx_v], add=True)
        return carry

    lax.fori_loop(0, EPW // SC_CHUNK, chunk, 0)
    plsc.subcore_barrier()
    pltpu.sync_copy(acc_sh.at[pl.ds(sid * NPT, NPT)],
                    out_hbm.at[cid].at[pl.ds(sid * NPT, NPT)])

    @pl.when(sid == 0)
    def _():
        pltpu.sync_copy(acc_sh.at[pl.ds(NPT * NS, NTAIL)],
                        out_hbm.at[cid].at[pl.ds(NPT * NS, NTAIL)])


def kernel(h_V, h_E, src_idx, batch_id, dst_idx, W1_w, W1_b, W2_w, W2_b, W3_w,
           W3_b, A, ln1_g, ln1_b, ln2_g, ln2_b, Win_w, Win_b, Wout_w, Wout_b):
    f32 = jnp.float32
    wp = W1_w[0:H]
    wq = W1_w[2 * H:3 * H]
    w1e = W1_w[H:2 * H]
    a2 = jnp.concatenate([A[0:H], A[2 * H:3 * H]], axis=1).T
    ae = A[H:2 * H].reshape(1, H)

    p_tab, q_tab, as_tab, ad_tab = pl.pallas_call(
        _prep_body,
        out_shape=[jax.ShapeDtypeStruct((N, H), f32),
                   jax.ShapeDtypeStruct((N, H), f32),
                   jax.ShapeDtypeStruct((1, N), f32),
                   jax.ShapeDtypeStruct((1, N), f32)],
    )(h_V, wp, wq, a2)

    sc_mesh = plsc.VectorSubcoreMesh(core_axis_name="c", subcore_axis_name="s")
    sc_params = pltpu.CompilerParams(needs_layout_passes=False)
    g1_tab, g2_tab, l_tab = pl.kernel(
        _sc_gather_body,
        compiler_params=sc_params,
        out_type=[jax.ShapeDtypeStruct((E, H), f32),
                  jax.ShapeDtypeStruct((E, H), f32),
                  jax.ShapeDtypeStruct((E,), f32)],
        mesh=sc_mesh,
        scratch_types=[
            pltpu.VMEM((GC,), jnp.int32),
            pltpu.VMEM((GC,), jnp.int32),
            pltpu.VMEM((GC, H), f32),
            pltpu.VMEM((GC, H), f32),
            pltpu.VMEM((N,), f32),
            pltpu.VMEM((N,), f32),
            pltpu.VMEM((GC,), f32),
            pltpu.SemaphoreType.DMA,
            pltpu.SemaphoreType.DMA,
        ],
    )(p_tab, q_tab, as_tab, ad_tab, src_idx, dst_idx)

    nb_e = E // EB
    msg, wsum = pl.pallas_call(
        _edge_body,
        grid=(nb_e,),
        in_specs=[
            pl.BlockSpec((EB, H), lambda i: (i, 0)),
            pl.BlockSpec((EB, H), lambda i: (i, 0)),
            pl.BlockSpec((EB, H), lambda i: (i, 0)),
            pl.BlockSpec((EB, 1), lambda i: (i, 0)),
            pl.BlockSpec((H, H), lambda i: (0, 0)),
            pl.BlockSpec((1, H), lambda i: (0, 0)),
            pl.BlockSpec((H, H), lambda i: (0, 0)),
            pl.BlockSpec((1, H), lambda i: (0, 0)),
            pl.BlockSpec((H, H), lambda i: (0, 0)),
            pl.BlockSpec((1, H), lambda i: (0, 0)),
            pl.BlockSpec((1, H), lambda i: (0, 0)),
        ],
        out_specs=[
            pl.BlockSpec((EB, H), lambda i: (i, 0)),
            pl.BlockSpec(memory_space=pltpu.SMEM),
        ],
        out_shape=[jax.ShapeDtypeStruct((E, H), f32),
                   jax.ShapeDtypeStruct((1, 1), f32)],
    )(h_E, g1_tab, g2_tab, l_tab.reshape(E, 1), w1e, W1_b.reshape(1, H), W2_w,
      W2_b.reshape(1, H), W3_w, W3_b.reshape(1, H), ae)

    parts = pl.kernel(
        _sc_scatter_body,
        compiler_params=sc_params,
        out_type=jax.ShapeDtypeStruct((NC, N, H), f32),
        mesh=sc_mesh,
        scratch_types=[
            pltpu.VMEM((SC_CHUNK,), jnp.int32),
            pltpu.VMEM((SC_CHUNK, H), f32),
            pltpu.VMEM_SHARED((N, H), f32),
            pltpu.SemaphoreType.DMA,
        ],
    )(msg, src_idx)

    nb_n = N // NB
    out = pl.pallas_call(
        _node_body,
        grid=(nb_n,),
        in_specs=[
            pl.BlockSpec((NB, H), lambda i: (i, 0)),
            pl.BlockSpec((NB, H), lambda i: (i, 0)),
            pl.BlockSpec((NB, H), lambda i: (i, 0)),
            pl.BlockSpec(memory_space=pltpu.SMEM),
            pl.BlockSpec((1, H), lambda i: (0, 0)),
            pl.BlockSpec((1, H), lambda i: (0, 0)),
            pl.BlockSpec((1, H), lambda i: (0, 0)),
            pl.BlockSpec((1, H), lambda i: (0, 0)),
            pl.BlockSpec((H, 4 * H), lambda i: (0, 0)),
            pl.BlockSpec((1, 4 * H), lambda i: (0, 0)),
            pl.BlockSpec((4 * H, H), lambda i: (0, 0)),
            pl.BlockSpec((1, H), lambda i: (0, 0)),
        ],
        out_specs=pl.BlockSpec((NB, H), lambda i: (i, 0)),
        out_shape=jax.ShapeDtypeStruct((N, H), f32),
    )(h_V, parts[0], parts[1], wsum, ln1_g.reshape(1, H), ln1_b.reshape(1, H),
      ln2_g.reshape(1, H), ln2_b.reshape(1, H), Win_w, Win_b.reshape(1, 4 * H),
      Wout_w, Wout_b.reshape(1, H))
    return out


# speedup vs baseline: 1.1573x; 1.1573x over previous
import jax
import jax.numpy as jnp
from jax import lax
from jax.experimental import pallas as pl
from jax.experimental.pallas import tpu as pltpu
from jax.experimental.pallas import tpu_sc as plsc

N = 10000
E = 320000
H = 128
NC = 2
NS = 16
NW = NC * NS
EPW = E // NW
GC = 80
SC_CHUNK = 200
NPT = 624
NTAIL = N - NPT * NS
EB = 2000
NB = 1000


def _gelu(x):
    return 0.5 * x * (1.0 + lax.erf(x * 0.7071067811865476))


def _prep_body(hv_ref, wp_ref, wq_ref, a2_ref, p_ref, q_ref, as_ref, ad_ref):
    hv = hv_ref[...]
    p_ref[...] = jnp.dot(hv, wp_ref[...], preferred_element_type=jnp.float32)
    q_ref[...] = jnp.dot(hv, wq_ref[...], preferred_element_type=jnp.float32)
    al = lax.dot_general(a2_ref[...], hv, (((1,), (1,)), ((), ())),
                         preferred_element_type=jnp.float32)
    as_ref[...] = al[0:1, :]
    ad_ref[...] = al[1:2, :]


def _edge_body(he_ref, g1_ref, g2_ref, lp_ref, w1e_ref, b1_ref, w2_ref,
               b2_ref, w3_ref, b3_ref, ae_ref, msg_ref, sum_ref):
    bf16 = jnp.bfloat16
    he = he_ref[...]
    x1 = (g1_ref[...] + g2_ref[...] + b1_ref[...]
          + jnp.dot(he.astype(bf16), w1e_ref[...].astype(bf16),
                    preferred_element_type=jnp.float32))
    x2 = jnp.dot(_gelu(x1).astype(bf16), w2_ref[...].astype(bf16),
                 preferred_element_type=jnp.float32) + b2_ref[...]
    msg = jnp.dot(_gelu(x2).astype(bf16), w3_ref[...].astype(bf16),
                  preferred_element_type=jnp.float32) + b3_ref[...]
    logit = lp_ref[...] + jnp.sum(he * ae_ref[...], axis=1, keepdims=True)
    leaky = jnp.where(logit >= 0, logit, 0.01 * logit)
    w = jnp.exp(1.0 / (1.0 + jnp.exp(-leaky)))

    @pl.when(pl.program_id(0) == 0)
    def _():
        sum_ref[0, 0] = 0.0

    sum_ref[0, 0] += jnp.sum(w)
    msg_ref[...] = msg * w


def _node_body(hv_ref, pa_ref, pb_ref, sum_ref, ln1g_ref, ln1b_ref, ln2g_ref,
               ln2b_ref, win_ref, winb_ref, wout_ref, woutb_ref, out_ref):
    scale = 1.0 / (30.0 * sum_ref[0, 0])
    x = hv_ref[...] + (pa_ref[...] + pb_ref[...]) * scale
    mu = jnp.mean(x, axis=1, keepdims=True)
    xc = x - mu
    var = jnp.mean(xc * xc, axis=1, keepdims=True)
    xn = xc * lax.rsqrt(var + 1e-5) * ln1g_ref[...] + ln1b_ref[...]
    y = jnp.dot(_gelu(jnp.dot(xn, win_ref[...], preferred_element_type=jnp.float32)
                      + winb_ref[...]),
                wout_ref[...], preferred_element_type=jnp.float32) + woutb_ref[...]
    z = xn + y
    mu2 = jnp.mean(z, axis=1, keepdims=True)
    zc = z - mu2
    var2 = jnp.mean(zc * zc, axis=1, keepdims=True)
    out_ref[...] = zc * lax.rsqrt(var2 + 1e-5) * ln2g_ref[...] + ln2b_ref[...]


def _sc_gather_body(p_hbm, q_hbm, as_hbm, ad_hbm, src_hbm, dst_hbm, g1_hbm,
                    g2_hbm, l_hbm, si_v, di_v, psA, qdA, psB, qdB, as_v, ad_v,
                    glA, glB, semA1, semA2, semB1, semB2):
    wid = lax.axis_index("s") * NC + lax.axis_index("c")
    base_w = wid * EPW
    pltpu.sync_copy(as_hbm.at[0], as_v)
    pltpu.sync_copy(ad_hbm.at[0], ad_v)
    pltpu.sync_copy(src_hbm.at[pl.ds(base_w, EPW)], si_v)
    pltpu.sync_copy(dst_hbm.at[pl.ds(base_w, EPW)], di_v)

    def logit(off, gl):
        def lrow(r, c2):
            sl = pl.ds(off + r * 16, 16)
            va = plsc.load_gather(as_v, [si_v[sl]])
            vb = plsc.load_gather(ad_v, [di_v[sl]])
            gl[pl.ds(r * 16, 16)] = va + vb
            return c2

        lax.fori_loop(0, GC // 16, lrow, 0)

    def drain(off, ps, qd, gl):
        pltpu.sync_copy(ps, g1_hbm.at[pl.ds(base_w + off, GC)])
        pltpu.sync_copy(qd, g2_hbm.at[pl.ds(base_w + off, GC)])
        pltpu.sync_copy(gl, l_hbm.at[pl.ds(base_w + off, GC)])

    def pair(i, carry):
        offA = 2 * GC * i
        offB = offA + GC
        cpA1 = pltpu.async_copy(p_hbm.at[si_v.at[pl.ds(offA, GC)]], psA, semA1)
        cpA2 = pltpu.async_copy(q_hbm.at[di_v.at[pl.ds(offA, GC)]], qdA, semA2)
        cpB1 = pltpu.async_copy(p_hbm.at[si_v.at[pl.ds(offB, GC)]], psB, semB1)
        cpB2 = pltpu.async_copy(q_hbm.at[di_v.at[pl.ds(offB, GC)]], qdB, semB2)
        logit(offA, glA)
        cpA1.wait()
        cpA2.wait()
        drain(offA, psA, qdA, glA)
        logit(offB, glB)
        cpB1.wait()
        cpB2.wait()
        drain(offB, psB, qdB, glB)
        return carry

    npairs = EPW // (2 * GC)
    lax.fori_loop(0, npairs, pair, 0)
    for off in range(npairs * 2 * GC, EPW, GC):
        cp1 = pltpu.async_copy(p_hbm.at[si_v.at[pl.ds(off, GC)]], psA, semA1)
        cp2 = pltpu.async_copy(q_hbm.at[di_v.at[pl.ds(off, GC)]], qdA, semA2)
        logit(off, glA)
        cp1.wait()
        cp2.wait()
        drain(off, psA, qdA, glA)


def _sc_scatter_body(msg_hbm, src_hbm, out_hbm, idx_v, msg_v, acc_sh, sem):
    cid = lax.axis_index("c")
    sid = lax.axis_index("s")
    wid = sid * NC + cid

    def zrow(r, c):
        for k in range(H // 16):
            msg_v[r, pl.ds(k * 16, 16)] = jnp.zeros((16,), jnp.float32)
        return c

    lax.fori_loop(0, SC_CHUNK, zrow, 0)
    for j in range(NPT // SC_CHUNK):
        pltpu.sync_copy(msg_v, acc_sh.at[pl.ds(sid * NPT + j * SC_CHUNK, SC_CHUNK)])
    pltpu.sync_copy(msg_v.at[pl.ds(0, NPT % SC_CHUNK)],
                    acc_sh.at[pl.ds(sid * NPT + (NPT // SC_CHUNK) * SC_CHUNK,
                                    NPT % SC_CHUNK)])

    @pl.when(sid == 0)
    def _():
        pltpu.sync_copy(msg_v.at[pl.ds(0, NTAIL)],
                        acc_sh.at[pl.ds(NPT * NS, NTAIL)])

    plsc.subcore_barrier()

    def chunk(i, carry):
        base = wid * EPW + i * SC_CHUNK
        pltpu.sync_copy(src_hbm.at[pl.ds(base, SC_CHUNK)], idx_v)
        pltpu.sync_copy(msg_hbm.at[pl.ds(base, SC_CHUNK)], msg_v)
        pltpu.sync_copy(msg_v, acc_sh.at[idx_v], add=True)
        return carry

    lax.fori_loop(0, EPW // SC_CHUNK, chunk, 0)
    plsc.subcore_barrier()
    pltpu.sync_copy(acc_sh.at[pl.ds(sid * NPT, NPT)],
                    out_hbm.at[cid].at[pl.ds(sid * NPT, NPT)])

    @pl.when(sid == 0)
    def _():
        pltpu.sync_copy(acc_sh.at[pl.ds(NPT * NS, NTAIL)],
                        out_hbm.at[cid].at[pl.ds(NPT * NS, NTAIL)])


def kernel(h_V, h_E, src_idx, batch_id, dst_idx, W1_w, W1_b, W2_w, W2_b, W3_w,
           W3_b, A, ln1_g, ln1_b, ln2_g, ln2_b, Win_w, Win_b, Wout_w, Wout_b):
    f32 = jnp.float32
    wp = W1_w[0:H]
    wq = W1_w[2 * H:3 * H]
    w1e = W1_w[H:2 * H]
    a2 = jnp.concatenate([A[0:H], A[2 * H:3 * H]], axis=1).T
    ae = A[H:2 * H].reshape(1, H)

    p_tab, q_tab, as_tab, ad_tab = pl.pallas_call(
        _prep_body,
        out_shape=[jax.ShapeDtypeStruct((N, H), f32),
                   jax.ShapeDtypeStruct((N, H), f32),
                   jax.ShapeDtypeStruct((1, N), f32),
                   jax.ShapeDtypeStruct((1, N), f32)],
    )(h_V, wp, wq, a2)

    sc_mesh = plsc.VectorSubcoreMesh(core_axis_name="c", subcore_axis_name="s")
    sc_params = pltpu.CompilerParams(needs_layout_passes=False)
    g1_tab, g2_tab, l_tab = pl.kernel(
        _sc_gather_body,
        compiler_params=sc_params,
        out_type=[jax.ShapeDtypeStruct((E, H), f32),
                  jax.ShapeDtypeStruct((E, H), f32),
                  jax.ShapeDtypeStruct((E,), f32)],
        mesh=sc_mesh,
        scratch_types=[
            pltpu.VMEM((EPW,), jnp.int32),
            pltpu.VMEM((EPW,), jnp.int32),
            pltpu.VMEM((GC, H), f32),
            pltpu.VMEM((GC, H), f32),
            pltpu.VMEM((GC, H), f32),
            pltpu.VMEM((GC, H), f32),
            pltpu.VMEM((N,), f32),
            pltpu.VMEM((N,), f32),
            pltpu.VMEM((GC,), f32),
            pltpu.VMEM((GC,), f32),
            pltpu.SemaphoreType.DMA,
            pltpu.SemaphoreType.DMA,
            pltpu.SemaphoreType.DMA,
            pltpu.SemaphoreType.DMA,
        ],
    )(p_tab, q_tab, as_tab, ad_tab, src_idx, dst_idx)

    nb_e = E // EB
    msg, wsum = pl.pallas_call(
        _edge_body,
        grid=(nb_e,),
        in_specs=[
            pl.BlockSpec((EB, H), lambda i: (i, 0)),
            pl.BlockSpec((EB, H), lambda i: (i, 0)),
            pl.BlockSpec((EB, H), lambda i: (i, 0)),
            pl.BlockSpec((EB, 1), lambda i: (i, 0)),
            pl.BlockSpec((H, H), lambda i: (0, 0)),
            pl.BlockSpec((1, H), lambda i: (0, 0)),
            pl.BlockSpec((H, H), lambda i: (0, 0)),
            pl.BlockSpec((1, H), lambda i: (0, 0)),
            pl.BlockSpec((H, H), lambda i: (0, 0)),
            pl.BlockSpec((1, H), lambda i: (0, 0)),
            pl.BlockSpec((1, H), lambda i: (0, 0)),
        ],
        out_specs=[
            pl.BlockSpec((EB, H), lambda i: (i, 0)),
            pl.BlockSpec(memory_space=pltpu.SMEM),
        ],
        out_shape=[jax.ShapeDtypeStruct((E, H), f32),
                   jax.ShapeDtypeStruct((1, 1), f32)],
    )(h_E, g1_tab, g2_tab, l_tab.reshape(E, 1), w1e, W1_b.reshape(1, H), W2_w,
      W2_b.reshape(1, H), W3_w, W3_b.reshape(1, H), ae)

    parts = pl.kernel(
        _sc_scatter_body,
        compiler_params=sc_params,
        out_type=jax.ShapeDtypeStruct((NC, N, H), f32),
        mesh=sc_mesh,
        scratch_types=[
            pltpu.VMEM((SC_CHUNK,), jnp.int32),
            pltpu.VMEM((SC_CHUNK, H), f32),
            pltpu.VMEM_SHARED((N, H), f32),
            pltpu.SemaphoreType.DMA,
        ],
    )(msg, src_idx)

    nb_n = N // NB
    out = pl.pallas_call(
        _node_body,
        grid=(nb_n,),
        in_specs=[
            pl.BlockSpec((NB, H), lambda i: (i, 0)),
            pl.BlockSpec((NB, H), lambda i: (i, 0)),
            pl.BlockSpec((NB, H), lambda i: (i, 0)),
            pl.BlockSpec(memory_space=pltpu.SMEM),
            pl.BlockSpec((1, H), lambda i: (0, 0)),
            pl.BlockSpec((1, H), lambda i: (0, 0)),
            pl.BlockSpec((1, H), lambda i: (0, 0)),
            pl.BlockSpec((1, H), lambda i: (0, 0)),
            pl.BlockSpec((H, 4 * H), lambda i: (0, 0)),
            pl.BlockSpec((1, 4 * H), lambda i: (0, 0)),
            pl.BlockSpec((4 * H, H), lambda i: (0, 0)),
            pl.BlockSpec((1, H), lambda i: (0, 0)),
        ],
        out_specs=pl.BlockSpec((NB, H), lambda i: (i, 0)),
        out_shape=jax.ShapeDtypeStruct((N, H), f32),
    )(h_V, parts[0], parts[1], wsum, ln1_g.reshape(1, H), ln1_b.reshape(1, H),
      ln2_g.reshape(1, H), ln2_b.reshape(1, H), Win_w, Win_b.reshape(1, 4 * H),
      Wout_w, Wout_b.reshape(1, H))
    return out


# speedup vs baseline: 1.1595x; 1.0019x over previous
import jax
import jax.numpy as jnp
from jax import lax
from jax.experimental import pallas as pl
from jax.experimental.pallas import tpu as pltpu
from jax.experimental.pallas import tpu_sc as plsc

N = 10000
E = 320000
H = 128
NC = 2
NS = 16
NW = NC * NS
ES = 64000
KSL = E // ES
EPWS = ES // NW
GC = 80
SC_CHUNK = 200
NPT = 624
NTAIL = N - NPT * NS
EB = 2000
NB = 1000


def _gelu(x):
    return 0.5 * x * (1.0 + lax.erf(x * 0.7071067811865476))


def _prep_body(hv_ref, wp_ref, wq_ref, a2_ref, p_ref, q_ref, as_ref, ad_ref):
    hv = hv_ref[...]
    p_ref[...] = jnp.dot(hv, wp_ref[...], preferred_element_type=jnp.float32)
    q_ref[...] = jnp.dot(hv, wq_ref[...], preferred_element_type=jnp.float32)
    al = lax.dot_general(a2_ref[...], hv, (((1,), (1,)), ((), ())),
                         preferred_element_type=jnp.float32)
    as_ref[...] = al[0:1, :]
    ad_ref[...] = al[1:2, :]


def _edge_body(he_ref, g1_ref, g2_ref, lp_ref, w1e_ref, b1_ref, w2_ref,
               b2_ref, w3_ref, b3_ref, ae_ref, msg_ref, sum_ref):
    bf16 = jnp.bfloat16
    he = he_ref[...]
    x1 = (g1_ref[...] + g2_ref[...] + b1_ref[...]
          + jnp.dot(he.astype(bf16), w1e_ref[...].astype(bf16),
                    preferred_element_type=jnp.float32))
    x2 = jnp.dot(_gelu(x1).astype(bf16), w2_ref[...].astype(bf16),
                 preferred_element_type=jnp.float32) + b2_ref[...]
    msg = jnp.dot(_gelu(x2).astype(bf16), w3_ref[...].astype(bf16),
                  preferred_element_type=jnp.float32) + b3_ref[...]
    logit = lp_ref[...] + jnp.sum(he * ae_ref[...], axis=1, keepdims=True)
    leaky = jnp.where(logit >= 0, logit, 0.01 * logit)
    w = jnp.exp(1.0 / (1.0 + jnp.exp(-leaky)))

    @pl.when(pl.program_id(0) == 0)
    def _():
        sum_ref[0, 0] = 0.0

    sum_ref[0, 0] += jnp.sum(w)
    msg_ref[...] = msg * w


def _node_body(hv_ref, pa_ref, pb_ref, sum_ref, ln1g_ref, ln1b_ref, ln2g_ref,
               ln2b_ref, win_ref, winb_ref, wout_ref, woutb_ref, out_ref):
    tot = sum_ref[0, 0]
    for k in range(1, KSL):
        tot += sum_ref[0, k]
    scale = 1.0 / (30.0 * tot)
    x = hv_ref[...] + (pa_ref[...] + pb_ref[...]) * scale
    mu = jnp.mean(x, axis=1, keepdims=True)
    xc = x - mu
    var = jnp.mean(xc * xc, axis=1, keepdims=True)
    xn = xc * lax.rsqrt(var + 1e-5) * ln1g_ref[...] + ln1b_ref[...]
    y = jnp.dot(_gelu(jnp.dot(xn, win_ref[...], preferred_element_type=jnp.float32)
                      + winb_ref[...]),
                wout_ref[...], preferred_element_type=jnp.float32) + woutb_ref[...]
    z = xn + y
    mu2 = jnp.mean(z, axis=1, keepdims=True)
    zc = z - mu2
    var2 = jnp.mean(zc * zc, axis=1, keepdims=True)
    out_ref[...] = zc * lax.rsqrt(var2 + 1e-5) * ln2g_ref[...] + ln2b_ref[...]


def _make_gather_body(k_off):
    def _sc_gather_body(p_hbm, q_hbm, as_hbm, ad_hbm, src_hbm, dst_hbm,
                        g1_hbm, g2_hbm, l_hbm, si_v, di_v, psA, qdA, psB, qdB,
                        as_v, ad_v, glA, glB, semA1, semA2, semB1, semB2):
        wid = lax.axis_index("s") * NC + lax.axis_index("c")
        base_w = wid * EPWS
        pltpu.sync_copy(as_hbm.at[0], as_v)
        pltpu.sync_copy(ad_hbm.at[0], ad_v)
        pltpu.sync_copy(src_hbm.at[pl.ds(k_off + base_w, EPWS)], si_v)
        pltpu.sync_copy(dst_hbm.at[pl.ds(k_off + base_w, EPWS)], di_v)

        def logit(off, gl):
            def lrow(r, c2):
                sl = pl.ds(off + r * 16, 16)
                va = plsc.load_gather(as_v, [si_v[sl]])
                vb = plsc.load_gather(ad_v, [di_v[sl]])
                gl[pl.ds(r * 16, 16)] = va + vb
                return c2

            lax.fori_loop(0, GC // 16, lrow, 0)

        def drain(off, ps, qd, gl):
            pltpu.sync_copy(ps, g1_hbm.at[pl.ds(base_w + off, GC)])
            pltpu.sync_copy(qd, g2_hbm.at[pl.ds(base_w + off, GC)])
            pltpu.sync_copy(gl, l_hbm.at[pl.ds(base_w + off, GC)])

        def pair(i, carry):
            offA = 2 * GC * i
            offB = offA + GC
            cpA1 = pltpu.async_copy(p_hbm.at[si_v.at[pl.ds(offA, GC)]],
                                    psA, semA1)
            cpA2 = pltpu.async_copy(q_hbm.at[di_v.at[pl.ds(offA, GC)]],
                                    qdA, semA2)
            cpB1 = pltpu.async_copy(p_hbm.at[si_v.at[pl.ds(offB, GC)]],
                                    psB, semB1)
            cpB2 = pltpu.async_copy(q_hbm.at[di_v.at[pl.ds(offB, GC)]],
                                    qdB, semB2)
            logit(offA, glA)
            cpA1.wait()
            cpA2.wait()
            drain(offA, psA, qdA, glA)
            logit(offB, glB)
            cpB1.wait()
            cpB2.wait()
            drain(offB, psB, qdB, glB)
            return carry

        npairs = EPWS // (2 * GC)
        lax.fori_loop(0, npairs, pair, 0)
        for off in range(npairs * 2 * GC, EPWS, GC):
            cp1 = pltpu.async_copy(p_hbm.at[si_v.at[pl.ds(off, GC)]],
                                   psA, semA1)
            cp2 = pltpu.async_copy(q_hbm.at[di_v.at[pl.ds(off, GC)]],
                                   qdA, semA2)
            logit(off, glA)
            cp1.wait()
            cp2.wait()
            drain(off, psA, qdA, glA)

    return _sc_gather_body


def _sc_scatter_body(m0, m1, m2, m3, m4, src_hbm, out_hbm, idx_v, msg_v,
                     acc_sh, sem):
    cid = lax.axis_index("c")
    sid = lax.axis_index("s")
    wid = sid * NC + cid
    msgs = (m0, m1, m2, m3, m4)

    def zrow(r, c):
        for k in range(H // 16):
            msg_v[r, pl.ds(k * 16, 16)] = jnp.zeros((16,), jnp.float32)
        return c

    lax.fori_loop(0, SC_CHUNK, zrow, 0)
    for j in range(NPT // SC_CHUNK):
        pltpu.sync_copy(msg_v, acc_sh.at[pl.ds(sid * NPT + j * SC_CHUNK, SC_CHUNK)])
    pltpu.sync_copy(msg_v.at[pl.ds(0, NPT % SC_CHUNK)],
                    acc_sh.at[pl.ds(sid * NPT + (NPT // SC_CHUNK) * SC_CHUNK,
                                    NPT % SC_CHUNK)])

    @pl.when(sid == 0)
    def _():
        pltpu.sync_copy(msg_v.at[pl.ds(0, NTAIL)],
                        acc_sh.at[pl.ds(NPT * NS, NTAIL)])

    plsc.subcore_barrier()

    for k in range(KSL):
        mk = msgs[k]

        def chunk(i, carry):
            base = wid * EPWS + i * SC_CHUNK
            pltpu.sync_copy(src_hbm.at[pl.ds(k * ES + base, SC_CHUNK)], idx_v)
            pltpu.sync_copy(mk.at[pl.ds(base, SC_CHUNK)], msg_v)
            pltpu.sync_copy(msg_v, acc_sh.at[idx_v], add=True)
            return carry

        lax.fori_loop(0, EPWS // SC_CHUNK, chunk, 0)
    plsc.subcore_barrier()
    pltpu.sync_copy(acc_sh.at[pl.ds(sid * NPT, NPT)],
                    out_hbm.at[cid].at[pl.ds(sid * NPT, NPT)])

    @pl.when(sid == 0)
    def _():
        pltpu.sync_copy(acc_sh.at[pl.ds(NPT * NS, NTAIL)],
                        out_hbm.at[cid].at[pl.ds(NPT * NS, NTAIL)])


def kernel(h_V, h_E, src_idx, batch_id, dst_idx, W1_w, W1_b, W2_w, W2_b, W3_w,
           W3_b, A, ln1_g, ln1_b, ln2_g, ln2_b, Win_w, Win_b, Wout_w, Wout_b):
    f32 = jnp.float32
    wp = W1_w[0:H]
    wq = W1_w[2 * H:3 * H]
    w1e = W1_w[H:2 * H]
    a2 = jnp.concatenate([A[0:H], A[2 * H:3 * H]], axis=1).T
    ae = A[H:2 * H].reshape(1, H)

    p_tab, q_tab, as_tab, ad_tab = pl.pallas_call(
        _prep_body,
        out_shape=[jax.ShapeDtypeStruct((N, H), f32),
                   jax.ShapeDtypeStruct((N, H), f32),
                   jax.ShapeDtypeStruct((1, N), f32),
                   jax.ShapeDtypeStruct((1, N), f32)],
    )(h_V, wp, wq, a2)

    sc_mesh = plsc.VectorSubcoreMesh(core_axis_name="c", subcore_axis_name="s")
    sc_params = pltpu.CompilerParams(needs_layout_passes=False)
    gather_scratch = [
        pltpu.VMEM((EPWS,), jnp.int32),
        pltpu.VMEM((EPWS,), jnp.int32),
        pltpu.VMEM((GC, H), f32),
        pltpu.VMEM((GC, H), f32),
        pltpu.VMEM((GC, H), f32),
        pltpu.VMEM((GC, H), f32),
        pltpu.VMEM((N,), f32),
        pltpu.VMEM((N,), f32),
        pltpu.VMEM((GC,), f32),
        pltpu.VMEM((GC,), f32),
        pltpu.SemaphoreType.DMA,
        pltpu.SemaphoreType.DMA,
        pltpu.SemaphoreType.DMA,
        pltpu.SemaphoreType.DMA,
    ]
    gathered = []
    for k in range(KSL):
        gathered.append(pl.kernel(
            _make_gather_body(k * ES),
            compiler_params=sc_params,
            out_type=[jax.ShapeDtypeStruct((ES, H), f32),
                      jax.ShapeDtypeStruct((ES, H), f32),
                      jax.ShapeDtypeStruct((ES,), f32)],
            mesh=sc_mesh,
            scratch_types=gather_scratch,
        )(p_tab, q_tab, as_tab, ad_tab, src_idx, dst_idx))

    nb_e = ES // EB
    msgs = []
    wsums = []
    for k in range(KSL):
        g1_s, g2_s, l_s = gathered[k]
        blk = k * nb_e
        msg_k, wsum_k = pl.pallas_call(
            _edge_body,
            grid=(nb_e,),
            in_specs=[
                pl.BlockSpec((EB, H), lambda i, b=blk: (i + b, 0)),
                pl.BlockSpec((EB, H), lambda i: (i, 0)),
                pl.BlockSpec((EB, H), lambda i: (i, 0)),
                pl.BlockSpec((EB, 1), lambda i: (i, 0)),
                pl.BlockSpec((H, H), lambda i: (0, 0)),
                pl.BlockSpec((1, H), lambda i: (0, 0)),
                pl.BlockSpec((H, H), lambda i: (0, 0)),
                pl.BlockSpec((1, H), lambda i: (0, 0)),
                pl.BlockSpec((H, H), lambda i: (0, 0)),
                pl.BlockSpec((1, H), lambda i: (0, 0)),
                pl.BlockSpec((1, H), lambda i: (0, 0)),
            ],
            out_specs=[
                pl.BlockSpec((EB, H), lambda i: (i, 0)),
                pl.BlockSpec(memory_space=pltpu.SMEM),
            ],
            out_shape=[jax.ShapeDtypeStruct((ES, H), f32),
                       jax.ShapeDtypeStruct((1, 1), f32)],
        )(h_E, g1_s, g2_s, l_s.reshape(ES, 1), w1e, W1_b.reshape(1, H), W2_w,
          W2_b.reshape(1, H), W3_w, W3_b.reshape(1, H), ae)
        msgs.append(msg_k)
        wsums.append(wsum_k)
    wsum = jnp.concatenate(wsums, axis=1)

    parts = pl.kernel(
        _sc_scatter_body,
        compiler_params=sc_params,
        out_type=jax.ShapeDtypeStruct((NC, N, H), f32),
        mesh=sc_mesh,
        scratch_types=[
            pltpu.VMEM((SC_CHUNK,), jnp.int32),
            pltpu.VMEM((SC_CHUNK, H), f32),
            pltpu.VMEM_SHARED((N, H), f32),
            pltpu.SemaphoreType.DMA,
        ],
    )(*msgs, src_idx)

    nb_n = N // NB
    out = pl.pallas_call(
        _node_body,
        grid=(nb_n,),
        in_specs=[
            pl.BlockSpec((NB, H), lambda i: (i, 0)),
            pl.BlockSpec((NB, H), lambda i: (i, 0)),
            pl.BlockSpec((NB, H), lambda i: (i, 0)),
            pl.BlockSpec(memory_space=pltpu.SMEM),
            pl.BlockSpec((1, H), lambda i: (0, 0)),
            pl.BlockSpec((1, H), lambda i: (0, 0)),
            pl.BlockSpec((1, H), lambda i: (0, 0)),
            pl.BlockSpec((1, H), lambda i: (0, 0)),
            pl.BlockSpec((H, 4 * H), lambda i: (0, 0)),
            pl.BlockSpec((1, 4 * H), lambda i: (0, 0)),
            pl.BlockSpec((4 * H, H), lambda i: (0, 0)),
            pl.BlockSpec((1, H), lambda i: (0, 0)),
        ],
        out_specs=pl.BlockSpec((NB, H), lambda i: (i, 0)),
        out_shape=jax.ShapeDtypeStruct((N, H), f32),
    )(h_V, parts[0], parts[1], wsum, ln1_g.reshape(1, H), ln1_b.reshape(1, H),
      ln2_g.reshape(1, H), ln2_b.reshape(1, H), Win_w, Win_b.reshape(1, 4 * H),
      Wout_w, Wout_b.reshape(1, H))
    return out


# speedup vs baseline: 1.2191x; 1.0514x over previous
import jax
import jax.numpy as jnp
from jax import lax
from jax.experimental import pallas as pl
from jax.experimental.pallas import tpu as pltpu
from jax.experimental.pallas import tpu_sc as plsc

N = 10000
E = 320000
H = 128
NC = 2
NS = 16
NW = NC * NS
ES = 64000
KSL = E // ES
EPWS = ES // NW
GC = 80
SC_CHUNK = 200
NPT = 624
NTAIL = N - NPT * NS
EB = 2000
NB = 1000


def _gelu(x):
    return 0.5 * x * (1.0 + lax.erf(x * 0.7071067811865476))


def _edge_body(he_ref, g1_ref, g2_ref, w1s_ref, w1d_ref, w1e_ref, b1_ref,
               w2_ref, b2_ref, w3_ref, b3_ref, as_ref, ad_ref, ae_ref,
               msg_ref, sum_ref):
    bf16 = jnp.bfloat16
    he = he_ref[...].astype(bf16)
    g1 = g1_ref[...].astype(bf16)
    g2 = g2_ref[...].astype(bf16)
    x1 = (jnp.dot(g1, w1s_ref[...].astype(bf16),
                  preferred_element_type=jnp.float32)
          + jnp.dot(g2, w1d_ref[...].astype(bf16),
                    preferred_element_type=jnp.float32)
          + jnp.dot(he, w1e_ref[...].astype(bf16),
                    preferred_element_type=jnp.float32)
          + b1_ref[...])
    x2 = jnp.dot(_gelu(x1).astype(bf16), w2_ref[...].astype(bf16),
                 preferred_element_type=jnp.float32) + b2_ref[...]
    msg = jnp.dot(_gelu(x2).astype(bf16), w3_ref[...].astype(bf16),
                  preferred_element_type=jnp.float32) + b3_ref[...]
    logit = (jnp.dot(g1, as_ref[...].astype(bf16),
                     preferred_element_type=jnp.float32)
             + jnp.dot(g2, ad_ref[...].astype(bf16),
                       preferred_element_type=jnp.float32)
             + jnp.dot(he, ae_ref[...].astype(bf16),
                       preferred_element_type=jnp.float32))
    leaky = jnp.where(logit >= 0, logit, 0.01 * logit)
    w = jnp.exp(1.0 / (1.0 + jnp.exp(-leaky)))

    @pl.when(pl.program_id(0) == 0)
    def _():
        sum_ref[0, 0] = 0.0

    sum_ref[0, 0] += jnp.sum(w)
    msg_ref[...] = msg * w


def _node_body(hv_ref, pa_ref, pb_ref, sum_ref, ln1g_ref, ln1b_ref, ln2g_ref,
               ln2b_ref, win_ref, winb_ref, wout_ref, woutb_ref, out_ref):
    tot = sum_ref[0, 0]
    for k in range(1, KSL):
        tot += sum_ref[0, k]
    scale = 1.0 / (30.0 * tot)
    x = hv_ref[...] + (pa_ref[...] + pb_ref[...]) * scale
    mu = jnp.mean(x, axis=1, keepdims=True)
    xc = x - mu
    var = jnp.mean(xc * xc, axis=1, keepdims=True)
    xn = xc * lax.rsqrt(var + 1e-5) * ln1g_ref[...] + ln1b_ref[...]
    y = jnp.dot(_gelu(jnp.dot(xn, win_ref[...], preferred_element_type=jnp.float32)
                      + winb_ref[...]),
                wout_ref[...], preferred_element_type=jnp.float32) + woutb_ref[...]
    z = xn + y
    mu2 = jnp.mean(z, axis=1, keepdims=True)
    zc = z - mu2
    var2 = jnp.mean(zc * zc, axis=1, keepdims=True)
    out_ref[...] = zc * lax.rsqrt(var2 + 1e-5) * ln2g_ref[...] + ln2b_ref[...]


def _make_gather_body(k_off):
    def _sc_gather_body(hv_hbm, src_hbm, dst_hbm, g1_hbm, g2_hbm, si_v, di_v,
                        psA, qdA, psB, qdB, semA1, semA2, semB1, semB2):
        wid = lax.axis_index("s") * NC + lax.axis_index("c")
        base_w = wid * EPWS
        pltpu.sync_copy(src_hbm.at[pl.ds(k_off + base_w, EPWS)], si_v)
        pltpu.sync_copy(dst_hbm.at[pl.ds(k_off + base_w, EPWS)], di_v)

        def drain(off, ps, qd):
            pltpu.sync_copy(ps, g1_hbm.at[pl.ds(base_w + off, GC)])
            pltpu.sync_copy(qd, g2_hbm.at[pl.ds(base_w + off, GC)])

        def pair(i, carry):
            offA = 2 * GC * i
            offB = offA + GC
            cpA1 = pltpu.async_copy(hv_hbm.at[si_v.at[pl.ds(offA, GC)]],
                                    psA, semA1)
            cpA2 = pltpu.async_copy(hv_hbm.at[di_v.at[pl.ds(offA, GC)]],
                                    qdA, semA2)
            cpB1 = pltpu.async_copy(hv_hbm.at[si_v.at[pl.ds(offB, GC)]],
                                    psB, semB1)
            cpB2 = pltpu.async_copy(hv_hbm.at[di_v.at[pl.ds(offB, GC)]],
                                    qdB, semB2)
            cpA1.wait()
            cpA2.wait()
            drain(offA, psA, qdA)
            cpB1.wait()
            cpB2.wait()
            drain(offB, psB, qdB)
            return carry

        npairs = EPWS // (2 * GC)
        lax.fori_loop(0, npairs, pair, 0)
        for off in range(npairs * 2 * GC, EPWS, GC):
            cp1 = pltpu.async_copy(hv_hbm.at[si_v.at[pl.ds(off, GC)]],
                                   psA, semA1)
            cp2 = pltpu.async_copy(hv_hbm.at[di_v.at[pl.ds(off, GC)]],
                                   qdA, semA2)
            cp1.wait()
            cp2.wait()
            drain(off, psA, qdA)

    return _sc_gather_body


def _sc_scatter_body(m0, m1, m2, m3, m4, src_hbm, out_hbm, idx_v, msg_v,
                     acc_sh, sem):
    cid = lax.axis_index("c")
    sid = lax.axis_index("s")
    wid = sid * NC + cid
    msgs = (m0, m1, m2, m3, m4)

    def zrow(r, c):
        for k in range(H // 16):
            msg_v[r, pl.ds(k * 16, 16)] = jnp.zeros((16,), jnp.float32)
        return c

    lax.fori_loop(0, SC_CHUNK, zrow, 0)
    for j in range(NPT // SC_CHUNK):
        pltpu.sync_copy(msg_v, acc_sh.at[pl.ds(sid * NPT + j * SC_CHUNK, SC_CHUNK)])
    pltpu.sync_copy(msg_v.at[pl.ds(0, NPT % SC_CHUNK)],
                    acc_sh.at[pl.ds(sid * NPT + (NPT // SC_CHUNK) * SC_CHUNK,
                                    NPT % SC_CHUNK)])

    @pl.when(sid == 0)
    def _():
        pltpu.sync_copy(msg_v.at[pl.ds(0, NTAIL)],
                        acc_sh.at[pl.ds(NPT * NS, NTAIL)])

    plsc.subcore_barrier()

    for k in range(KSL):
        mk = msgs[k]

        def chunk(i, carry):
            base = wid * EPWS + i * SC_CHUNK
            pltpu.sync_copy(src_hbm.at[pl.ds(k * ES + base, SC_CHUNK)], idx_v)
            pltpu.sync_copy(mk.at[pl.ds(base, SC_CHUNK)], msg_v)
            pltpu.sync_copy(msg_v, acc_sh.at[idx_v], add=True)
            return carry

        lax.fori_loop(0, EPWS // SC_CHUNK, chunk, 0)
    plsc.subcore_barrier()
    pltpu.sync_copy(acc_sh.at[pl.ds(sid * NPT, NPT)],
                    out_hbm.at[cid].at[pl.ds(sid * NPT, NPT)])

    @pl.when(sid == 0)
    def _():
        pltpu.sync_copy(acc_sh.at[pl.ds(NPT * NS, NTAIL)],
                        out_hbm.at[cid].at[pl.ds(NPT * NS, NTAIL)])


def kernel(h_V, h_E, src_idx, batch_id, dst_idx, W1_w, W1_b, W2_w, W2_b, W3_w,
           W3_b, A, ln1_g, ln1_b, ln2_g, ln2_b, Win_w, Win_b, Wout_w, Wout_b):
    f32 = jnp.float32
    w1s = W1_w[0:H]
    w1e = W1_w[H:2 * H]
    w1d = W1_w[2 * H:3 * H]
    a_s = A[0:H]
    a_e = A[H:2 * H]
    a_d = A[2 * H:3 * H]

    sc_mesh = plsc.VectorSubcoreMesh(core_axis_name="c", subcore_axis_name="s")
    sc_params = pltpu.CompilerParams(needs_layout_passes=False)
    gather_scratch = [
        pltpu.VMEM((EPWS,), jnp.int32),
        pltpu.VMEM((EPWS,), jnp.int32),
        pltpu.VMEM((GC, H), f32),
        pltpu.VMEM((GC, H), f32),
        pltpu.VMEM((GC, H), f32),
        pltpu.VMEM((GC, H), f32),
        pltpu.SemaphoreType.DMA,
        pltpu.SemaphoreType.DMA,
        pltpu.SemaphoreType.DMA,
        pltpu.SemaphoreType.DMA,
    ]
    gathered = []
    for k in range(KSL):
        gathered.append(pl.kernel(
            _make_gather_body(k * ES),
            compiler_params=sc_params,
            out_type=[jax.ShapeDtypeStruct((ES, H), f32),
                      jax.ShapeDtypeStruct((ES, H), f32)],
            mesh=sc_mesh,
            scratch_types=gather_scratch,
        )(h_V, src_idx, dst_idx))

    nb_e = ES // EB
    msgs = []
    wsums = []
    for k in range(KSL):
        g1_s, g2_s = gathered[k]
        blk = k * nb_e
        msg_k, wsum_k = pl.pallas_call(
            _edge_body,
            grid=(nb_e,),
            in_specs=[
                pl.BlockSpec((EB, H), lambda i, b=blk: (i + b, 0)),
                pl.BlockSpec((EB, H), lambda i: (i, 0)),
                pl.BlockSpec((EB, H), lambda i: (i, 0)),
                pl.BlockSpec((H, H), lambda i: (0, 0)),
                pl.BlockSpec((H, H), lambda i: (0, 0)),
                pl.BlockSpec((H, H), lambda i: (0, 0)),
                pl.BlockSpec((1, H), lambda i: (0, 0)),
                pl.BlockSpec((H, H), lambda i: (0, 0)),
                pl.BlockSpec((1, H), lambda i: (0, 0)),
                pl.BlockSpec((H, H), lambda i: (0, 0)),
                pl.BlockSpec((1, H), lambda i: (0, 0)),
                pl.BlockSpec((H, 1), lambda i: (0, 0)),
                pl.BlockSpec((H, 1), lambda i: (0, 0)),
                pl.BlockSpec((H, 1), lambda i: (0, 0)),
            ],
            out_specs=[
                pl.BlockSpec((EB, H), lambda i: (i, 0)),
                pl.BlockSpec(memory_space=pltpu.SMEM),
            ],
            out_shape=[jax.ShapeDtypeStruct((ES, H), f32),
                       jax.ShapeDtypeStruct((1, 1), f32)],
        )(h_E, g1_s, g2_s, w1s, w1d, w1e, W1_b.reshape(1, H), W2_w,
          W2_b.reshape(1, H), W3_w, W3_b.reshape(1, H), a_s, a_d, a_e)
        msgs.append(msg_k)
        wsums.append(wsum_k)
    wsum = jnp.concatenate(wsums, axis=1)

    parts = pl.kernel(
        _sc_scatter_body,
        compiler_params=sc_params,
        out_type=jax.ShapeDtypeStruct((NC, N, H), f32),
        mesh=sc_mesh,
        scratch_types=[
            pltpu.VMEM((SC_CHUNK,), jnp.int32),
            pltpu.VMEM((SC_CHUNK, H), f32),
            pltpu.VMEM_SHARED((N, H), f32),
            pltpu.SemaphoreType.DMA,
        ],
    )(*msgs, src_idx)

    nb_n = N // NB
    out = pl.pallas_call(
        _node_body,
        grid=(nb_n,),
        in_specs=[
            pl.BlockSpec((NB, H), lambda i: (i, 0)),
            pl.BlockSpec((NB, H), lambda i: (i, 0)),
            pl.BlockSpec((NB, H), lambda i: (i, 0)),
            pl.BlockSpec(memory_space=pltpu.SMEM),
            pl.BlockSpec((1, H), lambda i: (0, 0)),
            pl.BlockSpec((1, H), lambda i: (0, 0)),
            pl.BlockSpec((1, H), lambda i: (0, 0)),
            pl.BlockSpec((1, H), lambda i: (0, 0)),
            pl.BlockSpec((H, 4 * H), lambda i: (0, 0)),
            pl.BlockSpec((1, 4 * H), lambda i: (0, 0)),
            pl.BlockSpec((4 * H, H), lambda i: (0, 0)),
            pl.BlockSpec((1, H), lambda i: (0, 0)),
        ],
        out_specs=pl.BlockSpec((NB, H), lambda i: (i, 0)),
        out_shape=jax.ShapeDtypeStruct((N, H), f32),
    )(h_V, parts[0], parts[1], wsum, ln1_g.reshape(1, H), ln1_b.reshape(1, H),
      ln2_g.reshape(1, H), ln2_b.reshape(1, H), Win_w, Win_b.reshape(1, 4 * H),
      Wout_w, Wout_b.reshape(1, H))
    return out


# speedup vs baseline: 1.3426x; 1.1013x over previous
import jax
import jax.numpy as jnp
from jax import lax
from jax.experimental import pallas as pl
from jax.experimental.pallas import tpu as pltpu
from jax.experimental.pallas import tpu_sc as plsc

N = 10000
E = 320000
H = 128
NC = 2
NS = 16
NW = NC * NS
ES = 64000
KSL = E // ES
EPWS = ES // NW
GC = 80
SC_CHUNK = 200
NPT = 624
NTAIL = N - NPT * NS
EB = 2000
NB = 1000


def _gelu(x):
    return 0.5 * x * (1.0 + lax.erf(x * 0.7071067811865476))


def _edge_body(he_ref, g1_ref, g2_ref, w1s_ref, w1d_ref, w1e_ref, b1_ref,
               w2_ref, b2_ref, w3_ref, b3_ref, as_ref, ad_ref, ae_ref,
               msg_ref, sum_ref):
    bf16 = jnp.bfloat16
    he = he_ref[...].astype(bf16)
    g1 = g1_ref[...].astype(bf16)
    g2 = g2_ref[...].astype(bf16)
    x1 = (jnp.dot(g1, w1s_ref[...].astype(bf16),
                  preferred_element_type=jnp.float32)
          + jnp.dot(g2, w1d_ref[...].astype(bf16),
                    preferred_element_type=jnp.float32)
          + jnp.dot(he, w1e_ref[...].astype(bf16),
                    preferred_element_type=jnp.float32)
          + b1_ref[...])
    x2 = jnp.dot(_gelu(x1).astype(bf16), w2_ref[...].astype(bf16),
                 preferred_element_type=jnp.float32) + b2_ref[...]
    msg = jnp.dot(_gelu(x2).astype(bf16), w3_ref[...].astype(bf16),
                  preferred_element_type=jnp.float32) + b3_ref[...]
    logit = (jnp.dot(g1, as_ref[...].astype(bf16),
                     preferred_element_type=jnp.float32)
             + jnp.dot(g2, ad_ref[...].astype(bf16),
                       preferred_element_type=jnp.float32)
             + jnp.dot(he, ae_ref[...].astype(bf16),
                       preferred_element_type=jnp.float32))
    leaky = jnp.where(logit >= 0, logit, 0.01 * logit)
    w = jnp.exp(1.0 / (1.0 + jnp.exp(-leaky)))

    @pl.when(pl.program_id(0) == 0)
    def _():
        sum_ref[0, 0] = 0.0

    sum_ref[0, 0] += jnp.sum(w)
    msg_ref[...] = msg * w


def _node_body(hv_ref, pa_ref, pb_ref, pc_ref, pd_ref, sum_ref, ln1g_ref,
               ln1b_ref, ln2g_ref, ln2b_ref, win_ref, winb_ref, wout_ref,
               woutb_ref, out_ref):
    tot = sum_ref[0, 0]
    for k in range(1, KSL):
        tot += sum_ref[0, k]
    scale = 1.0 / (30.0 * tot)
    x = hv_ref[...] + ((pa_ref[...] + pb_ref[...])
                       + (pc_ref[...] + pd_ref[...])) * scale
    mu = jnp.mean(x, axis=1, keepdims=True)
    xc = x - mu
    var = jnp.mean(xc * xc, axis=1, keepdims=True)
    xn = xc * lax.rsqrt(var + 1e-5) * ln1g_ref[...] + ln1b_ref[...]
    y = jnp.dot(_gelu(jnp.dot(xn, win_ref[...], preferred_element_type=jnp.float32)
                      + winb_ref[...]),
                wout_ref[...], preferred_element_type=jnp.float32) + woutb_ref[...]
    z = xn + y
    mu2 = jnp.mean(z, axis=1, keepdims=True)
    zc = z - mu2
    var2 = jnp.mean(zc * zc, axis=1, keepdims=True)
    out_ref[...] = zc * lax.rsqrt(var2 + 1e-5) * ln2g_ref[...] + ln2b_ref[...]


def _make_gather_body(k_off):
    def _sc_gather_body(hv_hbm, src_hbm, dst_hbm, g1_hbm, g2_hbm, si_v, di_v,
                        psA, qdA, psB, qdB, semA1, semA2, semB1, semB2):
        wid = lax.axis_index("s") * NC + lax.axis_index("c")
        base_w = wid * EPWS
        pltpu.sync_copy(src_hbm.at[pl.ds(k_off + base_w, EPWS)], si_v)
        pltpu.sync_copy(dst_hbm.at[pl.ds(k_off + base_w, EPWS)], di_v)

        def drain(off, ps, qd):
            pltpu.sync_copy(ps, g1_hbm.at[pl.ds(base_w + off, GC)])
            pltpu.sync_copy(qd, g2_hbm.at[pl.ds(base_w + off, GC)])

        def pair(i, carry):
            offA = 2 * GC * i
            offB = offA + GC
            cpA1 = pltpu.async_copy(hv_hbm.at[si_v.at[pl.ds(offA, GC)]],
                                    psA, semA1)
            cpA2 = pltpu.async_copy(hv_hbm.at[di_v.at[pl.ds(offA, GC)]],
                                    qdA, semA2)
            cpB1 = pltpu.async_copy(hv_hbm.at[si_v.at[pl.ds(offB, GC)]],
                                    psB, semB1)
            cpB2 = pltpu.async_copy(hv_hbm.at[di_v.at[pl.ds(offB, GC)]],
                                    qdB, semB2)
            cpA1.wait()
            cpA2.wait()
            drain(offA, psA, qdA)
            cpB1.wait()
            cpB2.wait()
            drain(offB, psB, qdB)
            return carry

        npairs = EPWS // (2 * GC)
        lax.fori_loop(0, npairs, pair, 0)
        for off in range(npairs * 2 * GC, EPWS, GC):
            cp1 = pltpu.async_copy(hv_hbm.at[si_v.at[pl.ds(off, GC)]],
                                   psA, semA1)
            cp2 = pltpu.async_copy(hv_hbm.at[di_v.at[pl.ds(off, GC)]],
                                   qdA, semA2)
            cp1.wait()
            cp2.wait()
            drain(off, psA, qdA)

    return _sc_gather_body


def _scatter_impl(msgs, k0, src_hbm, out_hbm, idx_v, msg_v, acc_sh):
    cid = lax.axis_index("c")
    sid = lax.axis_index("s")
    wid = sid * NC + cid

    def zrow(r, c):
        for k in range(H // 16):
            msg_v[r, pl.ds(k * 16, 16)] = jnp.zeros((16,), jnp.float32)
        return c

    lax.fori_loop(0, SC_CHUNK, zrow, 0)
    for j in range(NPT // SC_CHUNK):
        pltpu.sync_copy(msg_v, acc_sh.at[pl.ds(sid * NPT + j * SC_CHUNK, SC_CHUNK)])
    pltpu.sync_copy(msg_v.at[pl.ds(0, NPT % SC_CHUNK)],
                    acc_sh.at[pl.ds(sid * NPT + (NPT // SC_CHUNK) * SC_CHUNK,
                                    NPT % SC_CHUNK)])

    @pl.when(sid == 0)
    def _():
        pltpu.sync_copy(msg_v.at[pl.ds(0, NTAIL)],
                        acc_sh.at[pl.ds(NPT * NS, NTAIL)])

    plsc.subcore_barrier()

    for k, mk in enumerate(msgs):
        goff = (k0 + k) * ES

        def chunk(i, carry):
            base = wid * EPWS + i * SC_CHUNK
            pltpu.sync_copy(src_hbm.at[pl.ds(goff + base, SC_CHUNK)], idx_v)
            pltpu.sync_copy(mk.at[pl.ds(base, SC_CHUNK)], msg_v)
            pltpu.sync_copy(msg_v, acc_sh.at[idx_v], add=True)
            return carry

        lax.fori_loop(0, EPWS // SC_CHUNK, chunk, 0)
    plsc.subcore_barrier()
    pltpu.sync_copy(acc_sh.at[pl.ds(sid * NPT, NPT)],
                    out_hbm.at[cid].at[pl.ds(sid * NPT, NPT)])

    @pl.when(sid == 0)
    def _():
        pltpu.sync_copy(acc_sh.at[pl.ds(NPT * NS, NTAIL)],
                        out_hbm.at[cid].at[pl.ds(NPT * NS, NTAIL)])


def _scatter_body_3(m0, m1, m2, src_hbm, out_hbm, idx_v, msg_v, acc_sh, sem):
    _scatter_impl((m0, m1, m2), 0, src_hbm, out_hbm, idx_v, msg_v, acc_sh)


def _scatter_body_2(m0, m1, src_hbm, out_hbm, idx_v, msg_v, acc_sh, sem):
    _scatter_impl((m0, m1), 3, src_hbm, out_hbm, idx_v, msg_v, acc_sh)


def kernel(h_V, h_E, src_idx, batch_id, dst_idx, W1_w, W1_b, W2_w, W2_b, W3_w,
           W3_b, A, ln1_g, ln1_b, ln2_g, ln2_b, Win_w, Win_b, Wout_w, Wout_b):
    f32 = jnp.float32
    w1s = W1_w[0:H]
    w1e = W1_w[H:2 * H]
    w1d = W1_w[2 * H:3 * H]
    a_s = A[0:H]
    a_e = A[H:2 * H]
    a_d = A[2 * H:3 * H]

    sc_mesh = plsc.VectorSubcoreMesh(core_axis_name="c", subcore_axis_name="s")
    sc_params = pltpu.CompilerParams(needs_layout_passes=False)
    gather_scratch = [
        pltpu.VMEM((EPWS,), jnp.int32),
        pltpu.VMEM((EPWS,), jnp.int32),
        pltpu.VMEM((GC, H), f32),
        pltpu.VMEM((GC, H), f32),
        pltpu.VMEM((GC, H), f32),
        pltpu.VMEM((GC, H), f32),
        pltpu.SemaphoreType.DMA,
        pltpu.SemaphoreType.DMA,
        pltpu.SemaphoreType.DMA,
        pltpu.SemaphoreType.DMA,
    ]
    gathered = []
    for k in range(KSL):
        gathered.append(pl.kernel(
            _make_gather_body(k * ES),
            compiler_params=sc_params,
            out_type=[jax.ShapeDtypeStruct((ES, H), f32),
                      jax.ShapeDtypeStruct((ES, H), f32)],
            mesh=sc_mesh,
            scratch_types=gather_scratch,
        )(h_V, src_idx, dst_idx))

    nb_e = ES // EB
    msgs = []
    wsums = []
    for k in range(KSL):
        g1_s, g2_s = gathered[k]
        blk = k * nb_e
        msg_k, wsum_k = pl.pallas_call(
            _edge_body,
            grid=(nb_e,),
            in_specs=[
                pl.BlockSpec((EB, H), lambda i, b=blk: (i + b, 0)),
                pl.BlockSpec((EB, H), lambda i: (i, 0)),
                pl.BlockSpec((EB, H), lambda i: (i, 0)),
                pl.BlockSpec((H, H), lambda i: (0, 0)),
                pl.BlockSpec((H, H), lambda i: (0, 0)),
                pl.BlockSpec((H, H), lambda i: (0, 0)),
                pl.BlockSpec((1, H), lambda i: (0, 0)),
                pl.BlockSpec((H, H), lambda i: (0, 0)),
                pl.BlockSpec((1, H), lambda i: (0, 0)),
                pl.BlockSpec((H, H), lambda i: (0, 0)),
                pl.BlockSpec((1, H), lambda i: (0, 0)),
                pl.BlockSpec((H, 1), lambda i: (0, 0)),
                pl.BlockSpec((H, 1), lambda i: (0, 0)),
                pl.BlockSpec((H, 1), lambda i: (0, 0)),
            ],
            out_specs=[
                pl.BlockSpec((EB, H), lambda i: (i, 0)),
                pl.BlockSpec(memory_space=pltpu.SMEM),
            ],
            out_shape=[jax.ShapeDtypeStruct((ES, H), f32),
                       jax.ShapeDtypeStruct((1, 1), f32)],
        )(h_E, g1_s, g2_s, w1s, w1d, w1e, W1_b.reshape(1, H), W2_w,
          W2_b.reshape(1, H), W3_w, W3_b.reshape(1, H), a_s, a_d, a_e)
        msgs.append(msg_k)
        wsums.append(wsum_k)
    wsum = jnp.concatenate(wsums, axis=1)

    scatter_scratch = [
        pltpu.VMEM((SC_CHUNK,), jnp.int32),
        pltpu.VMEM((SC_CHUNK, H), f32),
        pltpu.VMEM_SHARED((N, H), f32),
        pltpu.SemaphoreType.DMA,
    ]
    parts_a = pl.kernel(
        _scatter_body_3,
        compiler_params=sc_params,
        out_type=jax.ShapeDtypeStruct((NC, N, H), f32),
        mesh=sc_mesh,
        scratch_types=scatter_scratch,
    )(msgs[0], msgs[1], msgs[2], src_idx)
    parts_b = pl.kernel(
        _scatter_body_2,
        compiler_params=sc_params,
        out_type=jax.ShapeDtypeStruct((NC, N, H), f32),
        mesh=sc_mesh,
        scratch_types=scatter_scratch,
    )(msgs[3], msgs[4], src_idx)

    nb_n = N // NB
    out = pl.pallas_call(
        _node_body,
        grid=(nb_n,),
        in_specs=[
            pl.BlockSpec((NB, H), lambda i: (i, 0)),
            pl.BlockSpec((NB, H), lambda i: (i, 0)),
            pl.BlockSpec((NB, H), lambda i: (i, 0)),
            pl.BlockSpec((NB, H), lambda i: (i, 0)),
            pl.BlockSpec((NB, H), lambda i: (i, 0)),
            pl.BlockSpec(memory_space=pltpu.SMEM),
            pl.BlockSpec((1, H), lambda i: (0, 0)),
            pl.BlockSpec((1, H), lambda i: (0, 0)),
            pl.BlockSpec((1, H), lambda i: (0, 0)),
            pl.BlockSpec((1, H), lambda i: (0, 0)),
            pl.BlockSpec((H, 4 * H), lambda i: (0, 0)),
            pl.BlockSpec((1, 4 * H), lambda i: (0, 0)),
            pl.BlockSpec((4 * H, H), lambda i: (0, 0)),
            pl.BlockSpec((1, H), lambda i: (0, 0)),
        ],
        out_specs=pl.BlockSpec((NB, H), lambda i: (i, 0)),
        out_shape=jax.ShapeDtypeStruct((N, H), f32),
    )(h_V, parts_a[0], parts_a[1], parts_b[0], parts_b[1], wsum,
      ln1_g.reshape(1, H), ln1_b.reshape(1, H),
      ln2_g.reshape(1, H), ln2_b.reshape(1, H), Win_w, Win_b.reshape(1, 4 * H),
      Wout_w, Wout_b.reshape(1, H))
    return out


# speedup vs baseline: 1.6592x; 1.2359x over previous
import jax
import jax.numpy as jnp
from jax import lax
from jax.experimental import pallas as pl
from jax.experimental.pallas import tpu as pltpu
from jax.experimental.pallas import tpu_sc as plsc

N = 10000
E = 320000
H = 128
NC = 2
NS = 16
NW = NC * NS
ES = 64000
KSL = E // ES
EPWS = ES // NW
GC = 80
SC_CHUNK = 200
NPT = 624
NTAIL = N - NPT * NS
EB = 4000
NB = 1000


def _gelu(x):
    return 0.5 * x * (1.0 + lax.erf(x * 0.7071067811865476))


def _edge_body(he_ref, g1_ref, g2_ref, w1s_ref, w1d_ref, w1e_ref, b1_ref,
               w2_ref, b2_ref, w3_ref, b3_ref, as_ref, ad_ref, ae_ref,
               msg_ref, sum_ref):
    bf16 = jnp.bfloat16
    he = he_ref[...].astype(bf16)
    g1 = g1_ref[...].astype(bf16)
    g2 = g2_ref[...].astype(bf16)
    x1 = (jnp.dot(g1, w1s_ref[...].astype(bf16),
                  preferred_element_type=jnp.float32)
          + jnp.dot(g2, w1d_ref[...].astype(bf16),
                    preferred_element_type=jnp.float32)
          + jnp.dot(he, w1e_ref[...].astype(bf16),
                    preferred_element_type=jnp.float32)
          + b1_ref[...])
    x2 = jnp.dot(_gelu(x1).astype(bf16), w2_ref[...].astype(bf16),
                 preferred_element_type=jnp.float32) + b2_ref[...]
    msg = jnp.dot(_gelu(x2).astype(bf16), w3_ref[...].astype(bf16),
                  preferred_element_type=jnp.float32) + b3_ref[...]
    logit = (jnp.dot(g1, as_ref[...].astype(bf16),
                     preferred_element_type=jnp.float32)
             + jnp.dot(g2, ad_ref[...].astype(bf16),
                       preferred_element_type=jnp.float32)
             + jnp.dot(he, ae_ref[...].astype(bf16),
                       preferred_element_type=jnp.float32))
    leaky = jnp.where(logit >= 0, logit, 0.01 * logit)
    w = jnp.exp(1.0 / (1.0 + jnp.exp(-leaky)))

    @pl.when(pl.program_id(0) == 0)
    def _():
        sum_ref[0, 0] = 0.0

    sum_ref[0, 0] += jnp.sum(w)
    msg_ref[...] = msg * w


def _node_body(hv_ref, pa_ref, pb_ref, pc_ref, pd_ref, sum_ref, ln1g_ref,
               ln1b_ref, ln2g_ref, ln2b_ref, win_ref, winb_ref, wout_ref,
               woutb_ref, out_ref):
    tot = sum_ref[0, 0]
    for k in range(1, KSL):
        tot += sum_ref[0, k]
    scale = 1.0 / (30.0 * tot)
    x = hv_ref[...] + ((pa_ref[...] + pb_ref[...])
                       + (pc_ref[...] + pd_ref[...])) * scale
    mu = jnp.mean(x, axis=1, keepdims=True)
    xc = x - mu
    var = jnp.mean(xc * xc, axis=1, keepdims=True)
    xn = xc * lax.rsqrt(var + 1e-5) * ln1g_ref[...] + ln1b_ref[...]
    y = jnp.dot(_gelu(jnp.dot(xn, win_ref[...], preferred_element_type=jnp.float32)
                      + winb_ref[...]),
                wout_ref[...], preferred_element_type=jnp.float32) + woutb_ref[...]
    z = xn + y
    mu2 = jnp.mean(z, axis=1, keepdims=True)
    zc = z - mu2
    var2 = jnp.mean(zc * zc, axis=1, keepdims=True)
    out_ref[...] = zc * lax.rsqrt(var2 + 1e-5) * ln2g_ref[...] + ln2b_ref[...]


def _make_gather_body(k_off):
    def _sc_gather_body(hv_hbm, src_hbm, dst_hbm, g1_hbm, g2_hbm, si_v, di_v,
                        psA, qdA, psB, qdB, semA1, semA2, semB1, semB2):
        wid = lax.axis_index("s") * NC + lax.axis_index("c")
        base_w = wid * EPWS
        pltpu.sync_copy(src_hbm.at[pl.ds(k_off + base_w, EPWS)], si_v)
        pltpu.sync_copy(dst_hbm.at[pl.ds(k_off + base_w, EPWS)], di_v)

        def drain(off, ps, qd):
            pltpu.sync_copy(ps, g1_hbm.at[pl.ds(base_w + off, GC)])
            pltpu.sync_copy(qd, g2_hbm.at[pl.ds(base_w + off, GC)])

        def pair(i, carry):
            offA = 2 * GC * i
            offB = offA + GC
            cpA1 = pltpu.async_copy(hv_hbm.at[si_v.at[pl.ds(offA, GC)]],
                                    psA, semA1)
            cpA2 = pltpu.async_copy(hv_hbm.at[di_v.at[pl.ds(offA, GC)]],
                                    qdA, semA2)
            cpB1 = pltpu.async_copy(hv_hbm.at[si_v.at[pl.ds(offB, GC)]],
                                    psB, semB1)
            cpB2 = pltpu.async_copy(hv_hbm.at[di_v.at[pl.ds(offB, GC)]],
                                    qdB, semB2)
            cpA1.wait()
            cpA2.wait()
            drain(offA, psA, qdA)
            cpB1.wait()
            cpB2.wait()
            drain(offB, psB, qdB)
            return carry

        npairs = EPWS // (2 * GC)
        lax.fori_loop(0, npairs, pair, 0)
        for off in range(npairs * 2 * GC, EPWS, GC):
            cp1 = pltpu.async_copy(hv_hbm.at[si_v.at[pl.ds(off, GC)]],
                                   psA, semA1)
            cp2 = pltpu.async_copy(hv_hbm.at[di_v.at[pl.ds(off, GC)]],
                                   qdA, semA2)
            cp1.wait()
            cp2.wait()
            drain(off, psA, qdA)

    return _sc_gather_body


def _scatter_impl(msgs, k0, src_hbm, out_hbm, idx_v, msg_v, acc_sh):
    cid = lax.axis_index("c")
    sid = lax.axis_index("s")
    wid = sid * NC + cid

    def zrow(r, c):
        for k in range(H // 16):
            msg_v[r, pl.ds(k * 16, 16)] = jnp.zeros((16,), jnp.float32)
        return c

    lax.fori_loop(0, SC_CHUNK, zrow, 0)
    for j in range(NPT // SC_CHUNK):
        pltpu.sync_copy(msg_v, acc_sh.at[pl.ds(sid * NPT + j * SC_CHUNK, SC_CHUNK)])
    pltpu.sync_copy(msg_v.at[pl.ds(0, NPT % SC_CHUNK)],
                    acc_sh.at[pl.ds(sid * NPT + (NPT // SC_CHUNK) * SC_CHUNK,
                                    NPT % SC_CHUNK)])

    @pl.when(sid == 0)
    def _():
        pltpu.sync_copy(msg_v.at[pl.ds(0, NTAIL)],
                        acc_sh.at[pl.ds(NPT * NS, NTAIL)])

    plsc.subcore_barrier()

    for k, mk in enumerate(msgs):
        goff = (k0 + k) * ES

        def chunk(i, carry):
            base = wid * EPWS + i * SC_CHUNK
            pltpu.sync_copy(src_hbm.at[pl.ds(goff + base, SC_CHUNK)], idx_v)
            pltpu.sync_copy(mk.at[pl.ds(base, SC_CHUNK)], msg_v)
            pltpu.sync_copy(msg_v, acc_sh.at[idx_v], add=True)
            return carry

        lax.fori_loop(0, EPWS // SC_CHUNK, chunk, 0)
    plsc.subcore_barrier()
    pltpu.sync_copy(acc_sh.at[pl.ds(sid * NPT, NPT)],
                    out_hbm.at[cid].at[pl.ds(sid * NPT, NPT)])

    @pl.when(sid == 0)
    def _():
        pltpu.sync_copy(acc_sh.at[pl.ds(NPT * NS, NTAIL)],
                        out_hbm.at[cid].at[pl.ds(NPT * NS, NTAIL)])


def _scatter_body_3(m0, m1, m2, src_hbm, out_hbm, idx_v, msg_v, acc_sh, sem):
    _scatter_impl((m0, m1, m2), 0, src_hbm, out_hbm, idx_v, msg_v, acc_sh)


def _scatter_body_2(m0, m1, src_hbm, out_hbm, idx_v, msg_v, acc_sh, sem):
    _scatter_impl((m0, m1), 3, src_hbm, out_hbm, idx_v, msg_v, acc_sh)


def kernel(h_V, h_E, src_idx, batch_id, dst_idx, W1_w, W1_b, W2_w, W2_b, W3_w,
           W3_b, A, ln1_g, ln1_b, ln2_g, ln2_b, Win_w, Win_b, Wout_w, Wout_b):
    f32 = jnp.float32
    w1s = W1_w[0:H]
    w1e = W1_w[H:2 * H]
    w1d = W1_w[2 * H:3 * H]
    a_s = A[0:H]
    a_e = A[H:2 * H]
    a_d = A[2 * H:3 * H]

    sc_mesh = plsc.VectorSubcoreMesh(core_axis_name="c", subcore_axis_name="s")
    sc_params = pltpu.CompilerParams(needs_layout_passes=False)
    gather_scratch = [
        pltpu.VMEM((EPWS,), jnp.int32),
        pltpu.VMEM((EPWS,), jnp.int32),
        pltpu.VMEM((GC, H), f32),
        pltpu.VMEM((GC, H), f32),
        pltpu.VMEM((GC, H), f32),
        pltpu.VMEM((GC, H), f32),
        pltpu.SemaphoreType.DMA,
        pltpu.SemaphoreType.DMA,
        pltpu.SemaphoreType.DMA,
        pltpu.SemaphoreType.DMA,
    ]
    gathered = []
    for k in range(KSL):
        gathered.append(pl.kernel(
            _make_gather_body(k * ES),
            compiler_params=sc_params,
            out_type=[jax.ShapeDtypeStruct((ES, H), f32),
                      jax.ShapeDtypeStruct((ES, H), f32)],
            mesh=sc_mesh,
            scratch_types=gather_scratch,
        )(h_V, src_idx, dst_idx))

    nb_e = ES // EB
    msgs = []
    wsums = []
    for k in range(KSL):
        g1_s, g2_s = gathered[k]
        blk = k * nb_e
        msg_k, wsum_k = pl.pallas_call(
            _edge_body,
            grid=(nb_e,),
            in_specs=[
                pl.BlockSpec((EB, H), lambda i, b=blk: (i + b, 0)),
                pl.BlockSpec((EB, H), lambda i: (i, 0)),
                pl.BlockSpec((EB, H), lambda i: (i, 0)),
                pl.BlockSpec((H, H), lambda i: (0, 0)),
                pl.BlockSpec((H, H), lambda i: (0, 0)),
                pl.BlockSpec((H, H), lambda i: (0, 0)),
                pl.BlockSpec((1, H), lambda i: (0, 0)),
                pl.BlockSpec((H, H), lambda i: (0, 0)),
                pl.BlockSpec((1, H), lambda i: (0, 0)),
                pl.BlockSpec((H, H), lambda i: (0, 0)),
                pl.BlockSpec((1, H), lambda i: (0, 0)),
                pl.BlockSpec((H, 1), lambda i: (0, 0)),
                pl.BlockSpec((H, 1), lambda i: (0, 0)),
                pl.BlockSpec((H, 1), lambda i: (0, 0)),
            ],
            out_specs=[
                pl.BlockSpec((EB, H), lambda i: (i, 0)),
                pl.BlockSpec(memory_space=pltpu.SMEM),
            ],
            out_shape=[jax.ShapeDtypeStruct((ES, H), f32),
                       jax.ShapeDtypeStruct((1, 1), f32)],
        )(h_E, g1_s, g2_s, w1s, w1d, w1e, W1_b.reshape(1, H), W2_w,
          W2_b.reshape(1, H), W3_w, W3_b.reshape(1, H), a_s, a_d, a_e)
        msgs.append(msg_k)
        wsums.append(wsum_k)
    wsum = jnp.concatenate(wsums, axis=1)

    scatter_scratch = [
        pltpu.VMEM((SC_CHUNK,), jnp.int32),
        pltpu.VMEM((SC_CHUNK, H), f32),
        pltpu.VMEM_SHARED((N, H), f32),
        pltpu.SemaphoreType.DMA,
    ]
    parts_a = pl.kernel(
        _scatter_body_3,
        compiler_params=sc_params,
        out_type=jax.ShapeDtypeStruct((NC, N, H), f32),
        mesh=sc_mesh,
        scratch_types=scatter_scratch,
    )(msgs[0], msgs[1], msgs[2], src_idx)
    parts_b = pl.kernel(
        _scatter_body_2,
        compiler_params=sc_params,
        out_type=jax.ShapeDtypeStruct((NC, N, H), f32),
        mesh=sc_mesh,
        scratch_types=scatter_scratch,
    )(msgs[3], msgs[4], src_idx)

    nb_n = N // NB
    out = pl.pallas_call(
        _node_body,
        grid=(nb_n,),
        in_specs=[
            pl.BlockSpec((NB, H), lambda i: (i, 0)),
            pl.BlockSpec((NB, H), lambda i: (i, 0)),
            pl.BlockSpec((NB, H), lambda i: (i, 0)),
            pl.BlockSpec((NB, H), lambda i: (i, 0)),
            pl.BlockSpec((NB, H), lambda i: (i, 0)),
            pl.BlockSpec(memory_space=pltpu.SMEM),
            pl.BlockSpec((1, H), lambda i: (0, 0)),
            pl.BlockSpec((1, H), lambda i: (0, 0)),
            pl.BlockSpec((1, H), lambda i: (0, 0)),
            pl.BlockSpec((1, H), lambda i: (0, 0)),
            pl.BlockSpec((H, 4 * H), lambda i: (0, 0)),
            pl.BlockSpec((1, 4 * H), lambda i: (0, 0)),
            pl.BlockSpec((4 * H, H), lambda i: (0, 0)),
            pl.BlockSpec((1, H), lambda i: (0, 0)),
        ],
        out_specs=pl.BlockSpec((NB, H), lambda i: (i, 0)),
        out_shape=jax.ShapeDtypeStruct((N, H), f32),
    )(h_V, parts_a[0], parts_a[1], parts_b[0], parts_b[1], wsum,
      ln1_g.reshape(1, H), ln1_b.reshape(1, H),
      ln2_g.reshape(1, H), ln2_b.reshape(1, H), Win_w, Win_b.reshape(1, 4 * H),
      Wout_w, Wout_b.reshape(1, H))
    return out


# speedup vs baseline: 1.6678x; 1.0052x over previous
import jax
import jax.numpy as jnp
from jax import lax
from jax.experimental import pallas as pl
from jax.experimental.pallas import tpu as pltpu
from jax.experimental.pallas import tpu_sc as plsc

N = 10000
E = 320000
H = 128
NC = 2
NS = 16
NW = NC * NS
ES = 64000
KSL = E // ES
EPWS = ES // NW
GC = 80
SC_CHUNK = 200
NPT = 624
NTAIL = N - NPT * NS
EB = 8000
NB = 1000


def _gelu(x):
    return 0.5 * x * (1.0 + lax.erf(x * 0.7071067811865476))


def _edge_body(he_ref, g1_ref, g2_ref, w1s_ref, w1d_ref, w1e_ref, b1_ref,
               w2_ref, b2_ref, w3_ref, b3_ref, as_ref, ad_ref, ae_ref,
               msg_ref, sum_ref):
    bf16 = jnp.bfloat16
    he = he_ref[...].astype(bf16)
    g1 = g1_ref[...].astype(bf16)
    g2 = g2_ref[...].astype(bf16)
    x1 = (jnp.dot(g1, w1s_ref[...].astype(bf16),
                  preferred_element_type=jnp.float32)
          + jnp.dot(g2, w1d_ref[...].astype(bf16),
                    preferred_element_type=jnp.float32)
          + jnp.dot(he, w1e_ref[...].astype(bf16),
                    preferred_element_type=jnp.float32)
          + b1_ref[...])
    x2 = jnp.dot(_gelu(x1).astype(bf16), w2_ref[...].astype(bf16),
                 preferred_element_type=jnp.float32) + b2_ref[...]
    msg = jnp.dot(_gelu(x2).astype(bf16), w3_ref[...].astype(bf16),
                  preferred_element_type=jnp.float32) + b3_ref[...]
    logit = (jnp.dot(g1, as_ref[...].astype(bf16),
                     preferred_element_type=jnp.float32)
             + jnp.dot(g2, ad_ref[...].astype(bf16),
                       preferred_element_type=jnp.float32)
             + jnp.dot(he, ae_ref[...].astype(bf16),
                       preferred_element_type=jnp.float32))
    leaky = jnp.where(logit >= 0, logit, 0.01 * logit)
    w = jnp.exp(1.0 / (1.0 + jnp.exp(-leaky)))

    @pl.when(pl.program_id(0) == 0)
    def _():
        sum_ref[0, 0] = 0.0

    sum_ref[0, 0] += jnp.sum(w)
    msg_ref[...] = msg * w


def _node_body(hv_ref, pa_ref, pb_ref, pc_ref, pd_ref, sum_ref, ln1g_ref,
               ln1b_ref, ln2g_ref, ln2b_ref, win_ref, winb_ref, wout_ref,
               woutb_ref, out_ref):
    tot = sum_ref[0, 0]
    for k in range(1, KSL):
        tot += sum_ref[0, k]
    scale = 1.0 / (30.0 * tot)
    x = hv_ref[...] + ((pa_ref[...] + pb_ref[...])
                       + (pc_ref[...] + pd_ref[...])) * scale
    mu = jnp.mean(x, axis=1, keepdims=True)
    xc = x - mu
    var = jnp.mean(xc * xc, axis=1, keepdims=True)
    xn = xc * lax.rsqrt(var + 1e-5) * ln1g_ref[...] + ln1b_ref[...]
    y = jnp.dot(_gelu(jnp.dot(xn, win_ref[...], preferred_element_type=jnp.float32)
                      + winb_ref[...]),
                wout_ref[...], preferred_element_type=jnp.float32) + woutb_ref[...]
    z = xn + y
    mu2 = jnp.mean(z, axis=1, keepdims=True)
    zc = z - mu2
    var2 = jnp.mean(zc * zc, axis=1, keepdims=True)
    out_ref[...] = zc * lax.rsqrt(var2 + 1e-5) * ln2g_ref[...] + ln2b_ref[...]


def _make_gather_body(k_off):
    def _sc_gather_body(hv_hbm, src_hbm, dst_hbm, g1_hbm, g2_hbm, si_v, di_v,
                        psA, qdA, psB, qdB, semA1, semA2, semB1, semB2):
        wid = lax.axis_index("s") * NC + lax.axis_index("c")
        base_w = wid * EPWS
        pltpu.sync_copy(src_hbm.at[pl.ds(k_off + base_w, EPWS)], si_v)
        pltpu.sync_copy(dst_hbm.at[pl.ds(k_off + base_w, EPWS)], di_v)

        def drain(off, ps, qd):
            pltpu.sync_copy(ps, g1_hbm.at[pl.ds(base_w + off, GC)])
            pltpu.sync_copy(qd, g2_hbm.at[pl.ds(base_w + off, GC)])

        def pair(i, carry):
            offA = 2 * GC * i
            offB = offA + GC
            cpA1 = pltpu.async_copy(hv_hbm.at[si_v.at[pl.ds(offA, GC)]],
                                    psA, semA1)
            cpA2 = pltpu.async_copy(hv_hbm.at[di_v.at[pl.ds(offA, GC)]],
                                    qdA, semA2)
            cpB1 = pltpu.async_copy(hv_hbm.at[si_v.at[pl.ds(offB, GC)]],
                                    psB, semB1)
            cpB2 = pltpu.async_copy(hv_hbm.at[di_v.at[pl.ds(offB, GC)]],
                                    qdB, semB2)
            cpA1.wait()
            cpA2.wait()
            drain(offA, psA, qdA)
            cpB1.wait()
            cpB2.wait()
            drain(offB, psB, qdB)
            return carry

        npairs = EPWS // (2 * GC)
        lax.fori_loop(0, npairs, pair, 0)
        for off in range(npairs * 2 * GC, EPWS, GC):
            cp1 = pltpu.async_copy(hv_hbm.at[si_v.at[pl.ds(off, GC)]],
                                   psA, semA1)
            cp2 = pltpu.async_copy(hv_hbm.at[di_v.at[pl.ds(off, GC)]],
                                   qdA, semA2)
            cp1.wait()
            cp2.wait()
            drain(off, psA, qdA)

    return _sc_gather_body


def _scatter_impl(msgs, k0, src_hbm, out_hbm, idx_v, msg_v, acc_sh):
    cid = lax.axis_index("c")
    sid = lax.axis_index("s")
    wid = sid * NC + cid

    def zrow(r, c):
        for k in range(H // 16):
            msg_v[r, pl.ds(k * 16, 16)] = jnp.zeros((16,), jnp.float32)
        return c

    lax.fori_loop(0, SC_CHUNK, zrow, 0)
    for j in range(NPT // SC_CHUNK):
        pltpu.sync_copy(msg_v, acc_sh.at[pl.ds(sid * NPT + j * SC_CHUNK, SC_CHUNK)])
    pltpu.sync_copy(msg_v.at[pl.ds(0, NPT % SC_CHUNK)],
                    acc_sh.at[pl.ds(sid * NPT + (NPT // SC_CHUNK) * SC_CHUNK,
                                    NPT % SC_CHUNK)])

    @pl.when(sid == 0)
    def _():
        pltpu.sync_copy(msg_v.at[pl.ds(0, NTAIL)],
                        acc_sh.at[pl.ds(NPT * NS, NTAIL)])

    plsc.subcore_barrier()

    for k, mk in enumerate(msgs):
        goff = (k0 + k) * ES

        def chunk(i, carry):
            base = wid * EPWS + i * SC_CHUNK
            pltpu.sync_copy(src_hbm.at[pl.ds(goff + base, SC_CHUNK)], idx_v)
            pltpu.sync_copy(mk.at[pl.ds(base, SC_CHUNK)], msg_v)
            pltpu.sync_copy(msg_v, acc_sh.at[idx_v], add=True)
            return carry

        lax.fori_loop(0, EPWS // SC_CHUNK, chunk, 0)
    plsc.subcore_barrier()
    pltpu.sync_copy(acc_sh.at[pl.ds(sid * NPT, NPT)],
                    out_hbm.at[cid].at[pl.ds(sid * NPT, NPT)])

    @pl.when(sid == 0)
    def _():
        pltpu.sync_copy(acc_sh.at[pl.ds(NPT * NS, NTAIL)],
                        out_hbm.at[cid].at[pl.ds(NPT * NS, NTAIL)])


def _scatter_body_3(m0, m1, m2, src_hbm, out_hbm, idx_v, msg_v, acc_sh, sem):
    _scatter_impl((m0, m1, m2), 0, src_hbm, out_hbm, idx_v, msg_v, acc_sh)


def _scatter_body_2(m0, m1, src_hbm, out_hbm, idx_v, msg_v, acc_sh, sem):
    _scatter_impl((m0, m1), 3, src_hbm, out_hbm, idx_v, msg_v, acc_sh)


def kernel(h_V, h_E, src_idx, batch_id, dst_idx, W1_w, W1_b, W2_w, W2_b, W3_w,
           W3_b, A, ln1_g, ln1_b, ln2_g, ln2_b, Win_w, Win_b, Wout_w, Wout_b):
    f32 = jnp.float32
    w1s = W1_w[0:H]
    w1e = W1_w[H:2 * H]
    w1d = W1_w[2 * H:3 * H]
    a_s = A[0:H]
    a_e = A[H:2 * H]
    a_d = A[2 * H:3 * H]

    sc_mesh = plsc.VectorSubcoreMesh(core_axis_name="c", subcore_axis_name="s")
    sc_params = pltpu.CompilerParams(needs_layout_passes=False)
    gather_scratch = [
        pltpu.VMEM((EPWS,), jnp.int32),
        pltpu.VMEM((EPWS,), jnp.int32),
        pltpu.VMEM((GC, H), f32),
        pltpu.VMEM((GC, H), f32),
        pltpu.VMEM((GC, H), f32),
        pltpu.VMEM((GC, H), f32),
        pltpu.SemaphoreType.DMA,
        pltpu.SemaphoreType.DMA,
        pltpu.SemaphoreType.DMA,
        pltpu.SemaphoreType.DMA,
    ]
    gathered = []
    for k in range(KSL):
        gathered.append(pl.kernel(
            _make_gather_body(k * ES),
            compiler_params=sc_params,
            out_type=[jax.ShapeDtypeStruct((ES, H), f32),
                      jax.ShapeDtypeStruct((ES, H), f32)],
            mesh=sc_mesh,
            scratch_types=gather_scratch,
        )(h_V, src_idx, dst_idx))

    nb_e = ES // EB
    msgs = []
    wsums = []
    for k in range(KSL):
        g1_s, g2_s = gathered[k]
        blk = k * nb_e
        msg_k, wsum_k = pl.pallas_call(
            _edge_body,
            grid=(nb_e,),
            in_specs=[
                pl.BlockSpec((EB, H), lambda i, b=blk: (i + b, 0)),
                pl.BlockSpec((EB, H), lambda i: (i, 0)),
                pl.BlockSpec((EB, H), lambda i: (i, 0)),
                pl.BlockSpec((H, H), lambda i: (0, 0)),
                pl.BlockSpec((H, H), lambda i: (0, 0)),
                pl.BlockSpec((H, H), lambda i: (0, 0)),
                pl.BlockSpec((1, H), lambda i: (0, 0)),
                pl.BlockSpec((H, H), lambda i: (0, 0)),
                pl.BlockSpec((1, H), lambda i: (0, 0)),
                pl.BlockSpec((H, H), lambda i: (0, 0)),
                pl.BlockSpec((1, H), lambda i: (0, 0)),
                pl.BlockSpec((H, 1), lambda i: (0, 0)),
                pl.BlockSpec((H, 1), lambda i: (0, 0)),
                pl.BlockSpec((H, 1), lambda i: (0, 0)),
            ],
            out_specs=[
                pl.BlockSpec((EB, H), lambda i: (i, 0)),
                pl.BlockSpec(memory_space=pltpu.SMEM),
            ],
            out_shape=[jax.ShapeDtypeStruct((ES, H), f32),
                       jax.ShapeDtypeStruct((1, 1), f32)],
        )(h_E, g1_s, g2_s, w1s, w1d, w1e, W1_b.reshape(1, H), W2_w,
          W2_b.reshape(1, H), W3_w, W3_b.reshape(1, H), a_s, a_d, a_e)
        msgs.append(msg_k)
        wsums.append(wsum_k)
    wsum = jnp.concatenate(wsums, axis=1)

    scatter_scratch = [
        pltpu.VMEM((SC_CHUNK,), jnp.int32),
        pltpu.VMEM((SC_CHUNK, H), f32),
        pltpu.VMEM_SHARED((N, H), f32),
        pltpu.SemaphoreType.DMA,
    ]
    parts_a = pl.kernel(
        _scatter_body_3,
        compiler_params=sc_params,
        out_type=jax.ShapeDtypeStruct((NC, N, H), f32),
        mesh=sc_mesh,
        scratch_types=scatter_scratch,
    )(msgs[0], msgs[1], msgs[2], src_idx)
    parts_b = pl.kernel(
        _scatter_body_2,
        compiler_params=sc_params,
        out_type=jax.ShapeDtypeStruct((NC, N, H), f32),
        mesh=sc_mesh,
        scratch_types=scatter_scratch,
    )(msgs[3], msgs[4], src_idx)

    nb_n = N // NB
    out = pl.pallas_call(
        _node_body,
        grid=(nb_n,),
        in_specs=[
            pl.BlockSpec((NB, H), lambda i: (i, 0)),
            pl.BlockSpec((NB, H), lambda i: (i, 0)),
            pl.BlockSpec((NB, H), lambda i: (i, 0)),
            pl.BlockSpec((NB, H), lambda i: (i, 0)),
            pl.BlockSpec((NB, H), lambda i: (i, 0)),
            pl.BlockSpec(memory_space=pltpu.SMEM),
            pl.BlockSpec((1, H), lambda i: (0, 0)),
            pl.BlockSpec((1, H), lambda i: (0, 0)),
            pl.BlockSpec((1, H), lambda i: (0, 0)),
            pl.BlockSpec((1, H), lambda i: (0, 0)),
            pl.BlockSpec((H, 4 * H), lambda i: (0, 0)),
            pl.BlockSpec((1, 4 * H), lambda i: (0, 0)),
            pl.BlockSpec((4 * H, H), lambda i: (0, 0)),
            pl.BlockSpec((1, H), lambda i: (0, 0)),
        ],
        out_specs=pl.BlockSpec((NB, H), lambda i: (i, 0)),
        out_shape=jax.ShapeDtypeStruct((N, H), f32),
    )(h_V, parts_a[0], parts_a[1], parts_b[0], parts_b[1], wsum,
      ln1_g.reshape(1, H), ln1_b.reshape(1, H),
      ln2_g.reshape(1, H), ln2_b.reshape(1, H), Win_w, Win_b.reshape(1, 4 * H),
      Wout_w, Wout_b.reshape(1, H))
    return out
